# baseline (speedup 1.0000x reference)
import functools

import jax
import jax.numpy as jnp
from jax import lax
from jax.experimental import pallas as pl
from jax.experimental.pallas import tpu as pltpu
from jax.experimental.pallas import tpu_sc as plsc

VOCAB = 1000000
EMBED_DIM = 32
BATCH = 4096
HIST = 200
TOTAL = BATCH * HIST

NUM_CORES = 2
NUM_SUBCORES = 16
NW = NUM_CORES * NUM_SUBCORES
BBLK = BATCH // NW
HWIN = 5
NWIN = HIST // HWIN
ROWS = HWIN * BBLK
TP = TOTAL // (8 * 128)

_mesh = plsc.VectorSubcoreMesh(
    core_axis_name="c", subcore_axis_name="s",
    num_cores=NUM_CORES, num_subcores=NUM_SUBCORES)


TW = 896
NFULL = VOCAB // TW
TAILV = NFULL * TW
TAILW = VOCAB - TAILV


@functools.partial(
    pl.kernel,
    out_type=jax.ShapeDtypeStruct((VOCAB * EMBED_DIM,), jnp.float32),
    mesh=_mesh,
    scratch_types=[
        pltpu.VMEM((EMBED_DIM, TW), jnp.float32),
        pltpu.VMEM((EMBED_DIM, TW), jnp.float32),
        pltpu.VMEM((EMBED_DIM * TW,), jnp.float32),
        pltpu.VMEM((EMBED_DIM * TW,), jnp.float32),
        pltpu.SemaphoreType.DMA,
        pltpu.SemaphoreType.DMA,
    ],
    compiler_params=pltpu.CompilerParams(
        use_tc_tiling_on_sc=True, needs_layout_passes=False),
)
def _table_transpose(tv_hbm, out_hbm, in_v0, in_v1, tr_v0, tr_v1,
                     isem, osem):
    in_bufs = (in_v0, in_v1)
    tr_bufs = (tr_v0, tr_v1)
    wid = lax.axis_index("s") * NUM_CORES + lax.axis_index("c")
    iota16 = lax.iota(jnp.int32, 16)
    WORDS = EMBED_DIM * TW

    def load(v0, buf, width):
        v0 = pl.multiple_of(v0, 128)
        pltpu.async_copy(
            tv_hbm.at[:, pl.ds(v0, width)], in_bufs[buf], isem)

    def drain_in(buf, width):
        pltpu.make_async_copy(
            tv_hbm.at[:, pl.ds(0, width)], in_bufs[buf], isem).wait()

    def transpose(buf, width):
        @pl.loop(0, 16, unroll=2)
        def _d(d):
            x = (d + iota16) & 15
            xd = iota16 * EMBED_DIM + x
            for rt0 in range(0, width // 16, 4):
                group = [(rt0 + j, ct)
                         for j in range(4)
                         for ct in range(EMBED_DIM // 16)]
                vals = [
                    plsc.load_gather(
                        in_bufs[buf],
                        [x + 16 * ct, iota16 + 16 * rt])
                    for rt, ct in group
                ]
                for v, (rt, ct) in zip(vals, group):
                    dst = xd + (rt * 16 * EMBED_DIM + 16 * ct)
                    plsc.store_scatter(tr_bufs[buf], [dst], v)

    def writeback(v0, buf, width):
        pltpu.async_copy(
            tr_bufs[buf],
            out_hbm.at[pl.ds(v0 * EMBED_DIM, EMBED_DIM * width)], osem)

    def drain_out(buf, width):
        pltpu.make_async_copy(
            tr_bufs[buf],
            out_hbm.at[pl.ds(0, EMBED_DIM * width)], osem).wait()

    load(wid * TW, 0, TW)

    NI = (NFULL + NW - 1) // NW + 1

    @pl.loop(0, NI // 2)
    def _pair(pr):
        for t in range(2):
            i = 2 * pr + t
            win = wid + NW * i
            buf = t

            @pl.when(wid + NW * (i + 1) < NFULL)
            def _prefetch():
                load((wid + NW * (i + 1)) * TW, 1 - buf, TW)

            @pl.when(win < NFULL)
            def _body():
                drain_in(buf, TW)

                @pl.when(i >= 2)
                def _free():
                    drain_out(buf, TW)

                transpose(buf, TW)
                writeback(win * TW, buf, TW)

    drain_out(0, TW)
    drain_out(1, TW)


@functools.partial(
    pl.kernel,
    out_type=jax.ShapeDtypeStruct((TP * NW, 1024), jnp.float32),
    mesh=_mesh,
    scratch_types=[
        pltpu.VMEM((2, HWIN, BBLK), jnp.int32),
        pltpu.VMEM((ROWS, EMBED_DIM), jnp.float32),
        pltpu.VMEM((ROWS, EMBED_DIM), jnp.float32),
        pltpu.VMEM(((HWIN * EMBED_DIM // 8) * 1024,), jnp.float32),
        pltpu.VMEM(((HWIN * EMBED_DIM // 8) * 1024,), jnp.float32),
        pltpu.SemaphoreType.DMA,
        pltpu.SemaphoreType.DMA,
        pltpu.SemaphoreType.DMA,
    ],
    compiler_params=pltpu.CompilerParams(
        use_tc_tiling_on_sc=False, needs_layout_passes=False),
)
def _emb_lookup(idxT_hbm, table_hbm, out_hbm, idx_v, rows_v0, rows_v1,
                outv0, outv1, gsem, osem, isem):
    rows_bufs = (rows_v0, rows_v1)
    out_bufs = (outv0, outv1)
    wid = lax.axis_index("s") * NUM_CORES + lax.axis_index("c")
    b0 = wid * BBLK

    def load_idx(win, buf):
        pltpu.async_copy(
            idxT_hbm.at[pl.ds(win * HWIN, HWIN), pl.ds(b0, BBLK)],
            idx_v.at[buf], isem)

    def drain_idx(buf):
        pltpu.make_async_copy(
            idxT_hbm.at[pl.ds(0, HWIN), pl.ds(0, BBLK)],
            idx_v.at[buf], isem).wait()

    def fire_gathers(buf):
        for j in range(HWIN):
            for k in range(2):
                pltpu.async_copy(
                    table_hbm.at[idx_v.at[buf, j, pl.ds(64 * k, 64)]],
                    rows_bufs[buf].at[pl.ds(j * BBLK + 64 * k, 64)],
                    gsem,
                )

    def drain_gathers(buf):
        pltpu.make_async_copy(
            table_hbm.at[pl.ds(0, ROWS)], rows_bufs[buf], gsem).wait()

    iota16 = lax.iota(jnp.int32, 16)

    def transpose(buf):
        @pl.loop(0, 16, unroll=2)
        def _d(d):
            x = (d + iota16) & 15
            dst_x = ((x >> 3) << 10) + ((x & 7) << 7) + iota16
            for h in range(HWIN):
                for bt0 in range(0, BBLK // 16, 8):
                    group = [(bt0 + j, k0)
                             for j in range(8)
                             for k0 in range(EMBED_DIM // 16)]
                    vals = [
                        plsc.load_gather(
                            rows_bufs[buf],
                            [iota16 + (h * BBLK + 16 * bt), x + 16 * k0])
                        for bt, k0 in group
                    ]
                    for v, (bt, k0) in zip(vals, group):
                        dst = dst_x + ((h * 4 + 2 * k0) * 1024 + 16 * bt)
                        plsc.store_scatter(out_bufs[buf], [dst], v)

    WTP = HWIN * EMBED_DIM // 8

    def writeback(win, buf):
        for a in range(WTP):
            pltpu.async_copy(
                out_bufs[buf].at[pl.ds(a * 1024, 1024)],
                out_hbm.at[(win * WTP + a) * NW + wid],
                osem)

    def drain_out(buf):
        for a in range(WTP):
            pltpu.make_async_copy(
                out_bufs[buf].at[pl.ds(a * 1024, 1024)], out_hbm.at[0],
                osem).wait()

    load_idx(0, 0)
    drain_idx(0)
    fire_gathers(0)
    load_idx(1, 1)

    @pl.loop(0, NWIN // 2)
    def _pair(pr):
        w0 = 2 * pr
        for t in range(2):
            win = w0 + t
            buf = t
            nbuf = 1 - t

            @pl.when(win + 1 < NWIN)
            def _prefetch():
                drain_idx(nbuf)
                fire_gathers(nbuf)

            drain_gathers(buf)

            @pl.when(win + 2 < NWIN)
            def _iprefetch():
                load_idx(win + 2, buf)

            @pl.when(win >= 2)
            def _free():
                drain_out(buf)

            transpose(buf)
            writeback(win, buf)

    drain_out(0)
    drain_out(1)


@jax.jit
def kernel(w_tensor, table):
    idxT = w_tensor.astype(jnp.int32).T
    table_lin = _table_transpose(table.T)
    tail = table[TAILV:].reshape(TAILW * EMBED_DIM)
    table_lin = table_lin.at[TAILV * EMBED_DIM:].set(tail)
    table_rm = table_lin.reshape(VOCAB, EMBED_DIM)
    out2d = _emb_lookup(idxT, table_rm)
    out4d = out2d.reshape(TP, NW, 8, 128)
    out = out4d.transpose(1, 3, 0, 2).reshape(BATCH, HIST * EMBED_DIM)
    return out.reshape(BATCH, HIST, EMBED_DIM)


# speedup vs baseline: 1.0957x; 1.0957x over previous
import functools

import jax
import jax.numpy as jnp
from jax import lax
from jax.experimental import pallas as pl
from jax.experimental.pallas import tpu as pltpu
from jax.experimental.pallas import tpu_sc as plsc

VOCAB = 1000000
EMBED_DIM = 32
BATCH = 4096
HIST = 200
TOTAL = BATCH * HIST

NUM_CORES = 2
NUM_SUBCORES = 16
NW = NUM_CORES * NUM_SUBCORES
BBLK = BATCH // NW
HWIN = 5
NWIN = HIST // HWIN
ROWS = HWIN * BBLK
TP = TOTAL // (8 * 128)

_mesh = plsc.VectorSubcoreMesh(
    core_axis_name="c", subcore_axis_name="s",
    num_cores=NUM_CORES, num_subcores=NUM_SUBCORES)


TW = 896
NFULL = VOCAB // TW
TAILV = NFULL * TW
TAILW = VOCAB - TAILV


@functools.partial(
    pl.kernel,
    out_type=jax.ShapeDtypeStruct((VOCAB * EMBED_DIM,), jnp.float32),
    mesh=_mesh,
    scratch_types=[
        pltpu.VMEM((EMBED_DIM, TW), jnp.float32),
        pltpu.VMEM((EMBED_DIM, TW), jnp.float32),
        pltpu.VMEM((EMBED_DIM * TW,), jnp.float32),
        pltpu.VMEM((EMBED_DIM * TW,), jnp.float32),
        pltpu.SemaphoreType.DMA,
        pltpu.SemaphoreType.DMA,
    ],
    compiler_params=pltpu.CompilerParams(
        use_tc_tiling_on_sc=True, needs_layout_passes=False),
)
def _table_transpose(tv_hbm, out_hbm, in_v0, in_v1, tr_v0, tr_v1,
                     isem, osem):
    in_bufs = (in_v0, in_v1)
    tr_bufs = (tr_v0, tr_v1)
    wid = lax.axis_index("s") * NUM_CORES + lax.axis_index("c")
    iota16 = lax.iota(jnp.int32, 16)
    WORDS = EMBED_DIM * TW

    def load(v0, buf, width):
        v0 = pl.multiple_of(v0, 128)
        pltpu.async_copy(
            tv_hbm.at[:, pl.ds(v0, width)], in_bufs[buf], isem)

    def drain_in(buf, width):
        pltpu.make_async_copy(
            tv_hbm.at[:, pl.ds(0, width)], in_bufs[buf], isem).wait()

    def transpose(buf, width):
        @pl.loop(0, 16, unroll=2)
        def _d(d):
            x = (d + iota16) & 15
            xd = iota16 * EMBED_DIM + x
            for rt0 in range(0, width // 16, 4):
                group = [(rt0 + j, ct)
                         for j in range(4)
                         for ct in range(EMBED_DIM // 16)]
                vals = [
                    plsc.load_gather(
                        in_bufs[buf],
                        [x + 16 * ct, iota16 + 16 * rt])
                    for rt, ct in group
                ]
                for v, (rt, ct) in zip(vals, group):
                    dst = xd + (rt * 16 * EMBED_DIM + 16 * ct)
                    plsc.store_scatter(tr_bufs[buf], [dst], v)

    def writeback(v0, buf, width):
        pltpu.async_copy(
            tr_bufs[buf],
            out_hbm.at[pl.ds(v0 * EMBED_DIM, EMBED_DIM * width)], osem)

    def drain_out(buf, width):
        pltpu.make_async_copy(
            tr_bufs[buf],
            out_hbm.at[pl.ds(0, EMBED_DIM * width)], osem).wait()

    load(wid * TW, 0, TW)

    NI = (NFULL + NW - 1) // NW + 1

    @pl.loop(0, NI // 2)
    def _pair(pr):
        for t in range(2):
            i = 2 * pr + t
            win = wid + NW * i
            buf = t

            @pl.when(wid + NW * (i + 1) < NFULL)
            def _prefetch():
                load((wid + NW * (i + 1)) * TW, 1 - buf, TW)

            @pl.when(win < NFULL)
            def _body():
                drain_in(buf, TW)

                @pl.when(i >= 2)
                def _free():
                    drain_out(buf, TW)

                transpose(buf, TW)
                writeback(win * TW, buf, TW)

    drain_out(0, TW)
    drain_out(1, TW)


@functools.partial(
    pl.kernel,
    out_type=jax.ShapeDtypeStruct((TP * NW, 1024), jnp.float32),
    mesh=_mesh,
    scratch_types=[
        pltpu.VMEM((2, HWIN, BBLK), jnp.int32),
        pltpu.VMEM((ROWS, EMBED_DIM), jnp.float32),
        pltpu.VMEM((ROWS, EMBED_DIM), jnp.float32),
        pltpu.VMEM(((HWIN * EMBED_DIM // 8) * 1024,), jnp.float32),
        pltpu.VMEM(((HWIN * EMBED_DIM // 8) * 1024,), jnp.float32),
        pltpu.SemaphoreType.DMA,
        pltpu.SemaphoreType.DMA,
        pltpu.SemaphoreType.DMA,
    ],
    compiler_params=pltpu.CompilerParams(
        use_tc_tiling_on_sc=False, needs_layout_passes=False),
)
def _emb_lookup(idxT_hbm, table_hbm, out_hbm, idx_v, rows_v0, rows_v1,
                outv0, outv1, gsem, osem, isem):
    rows_bufs = (rows_v0, rows_v1)
    out_bufs = (outv0, outv1)
    wid = lax.axis_index("s") * NUM_CORES + lax.axis_index("c")
    b0 = wid * BBLK

    def load_idx(win, buf):
        pltpu.async_copy(
            idxT_hbm.at[pl.ds(win * HWIN, HWIN), pl.ds(b0, BBLK)],
            idx_v.at[buf], isem)

    def drain_idx(buf):
        pltpu.make_async_copy(
            idxT_hbm.at[pl.ds(0, HWIN), pl.ds(0, BBLK)],
            idx_v.at[buf], isem).wait()

    def fire_gathers(buf):
        for j in range(HWIN):
            for k in range(2):
                pltpu.async_copy(
                    table_hbm.at[idx_v.at[buf, j, pl.ds(64 * k, 64)]],
                    rows_bufs[buf].at[pl.ds(j * BBLK + 64 * k, 64)],
                    gsem,
                )

    def drain_gathers(buf):
        pltpu.make_async_copy(
            table_hbm.at[pl.ds(0, ROWS)], rows_bufs[buf], gsem).wait()

    iota16 = lax.iota(jnp.int32, 16)

    def transpose(buf):
        @pl.loop(0, 16, unroll=2)
        def _d(d):
            x = (d + iota16) & 15
            dst_x = ((x >> 3) << 10) + ((x & 7) << 7) + iota16
            for h in range(HWIN):
                for bt0 in range(0, BBLK // 16, 4):
                    group = [(bt0 + j, k0)
                             for j in range(4)
                             for k0 in range(EMBED_DIM // 16)]
                    vals = [
                        plsc.load_gather(
                            rows_bufs[buf],
                            [iota16 + (h * BBLK + 16 * bt), x + 16 * k0])
                        for bt, k0 in group
                    ]
                    for v, (bt, k0) in zip(vals, group):
                        dst = dst_x + ((h * 4 + 2 * k0) * 1024 + 16 * bt)
                        plsc.store_scatter(out_bufs[buf], [dst], v)

    WTP = HWIN * EMBED_DIM // 8

    def writeback(win, buf):
        for a in range(WTP):
            pltpu.async_copy(
                out_bufs[buf].at[pl.ds(a * 1024, 1024)],
                out_hbm.at[(win * WTP + a) * NW + wid],
                osem)

    def drain_out(buf):
        for a in range(WTP):
            pltpu.make_async_copy(
                out_bufs[buf].at[pl.ds(a * 1024, 1024)], out_hbm.at[0],
                osem).wait()

    load_idx(0, 0)
    drain_idx(0)
    fire_gathers(0)
    load_idx(1, 1)

    @pl.loop(0, NWIN // 2)
    def _pair(pr):
        w0 = 2 * pr
        for t in range(2):
            win = w0 + t
            buf = t
            nbuf = 1 - t

            @pl.when(win + 1 < NWIN)
            def _prefetch():
                drain_idx(nbuf)
                fire_gathers(nbuf)

            drain_gathers(buf)

            @pl.when(win + 2 < NWIN)
            def _iprefetch():
                load_idx(win + 2, buf)

            @pl.when(win >= 2)
            def _free():
                drain_out(buf)

            transpose(buf)
            writeback(win, buf)

    drain_out(0)
    drain_out(1)


@jax.jit
def kernel(w_tensor, table):
    idxT = w_tensor.astype(jnp.int32).T
    table_lin = _table_transpose(table.T)
    tail = table[TAILV:].reshape(TAILW * EMBED_DIM)
    table_lin = table_lin.at[TAILV * EMBED_DIM:].set(tail)
    table_rm = table_lin.reshape(VOCAB, EMBED_DIM)
    out2d = _emb_lookup(idxT, table_rm)
    out4d = out2d.reshape(TP, NW, 8, 128)
    out = out4d.transpose(1, 3, 0, 2).reshape(BATCH, HIST * EMBED_DIM)
    return out.reshape(BATCH, HIST, EMBED_DIM)


# speedup vs baseline: 1.0974x; 1.0015x over previous
import functools

import jax
import jax.numpy as jnp
from jax import lax
from jax.experimental import pallas as pl
from jax.experimental.pallas import tpu as pltpu
from jax.experimental.pallas import tpu_sc as plsc

VOCAB = 1000000
EMBED_DIM = 32
BATCH = 4096
HIST = 200
TOTAL = BATCH * HIST

NUM_CORES = 2
NUM_SUBCORES = 16
NW = NUM_CORES * NUM_SUBCORES
BBLK = BATCH // NW
HWIN = 5
NWIN = HIST // HWIN
ROWS = HWIN * BBLK
TP = TOTAL // (8 * 128)

_mesh = plsc.VectorSubcoreMesh(
    core_axis_name="c", subcore_axis_name="s",
    num_cores=NUM_CORES, num_subcores=NUM_SUBCORES)


TW = 896
NFULL = VOCAB // TW
TAILV = NFULL * TW
TAILW = VOCAB - TAILV


@functools.partial(
    pl.kernel,
    out_type=jax.ShapeDtypeStruct((VOCAB * EMBED_DIM,), jnp.float32),
    mesh=_mesh,
    scratch_types=[
        pltpu.VMEM((EMBED_DIM, TW), jnp.float32),
        pltpu.VMEM((EMBED_DIM, TW), jnp.float32),
        pltpu.VMEM((EMBED_DIM * TW,), jnp.float32),
        pltpu.VMEM((EMBED_DIM * TW,), jnp.float32),
        pltpu.SemaphoreType.DMA,
        pltpu.SemaphoreType.DMA,
    ],
    compiler_params=pltpu.CompilerParams(
        use_tc_tiling_on_sc=True, needs_layout_passes=False),
)
def _table_transpose(tv_hbm, out_hbm, in_v0, in_v1, tr_v0, tr_v1,
                     isem, osem):
    in_bufs = (in_v0, in_v1)
    tr_bufs = (tr_v0, tr_v1)
    wid = lax.axis_index("s") * NUM_CORES + lax.axis_index("c")
    iota16 = lax.iota(jnp.int32, 16)

    def load(v0, buf, width):
        v0 = pl.multiple_of(v0, 128)
        pltpu.async_copy(
            tv_hbm.at[:, pl.ds(v0, width)], in_bufs[buf], isem)

    def drain_in(buf, width):
        pltpu.make_async_copy(
            tv_hbm.at[:, pl.ds(0, width)], in_bufs[buf], isem).wait()

    def transpose(buf, width):
        @pl.loop(0, 16, unroll=2)
        def _d(d):
            x = (d + iota16) & 15
            xd = iota16 * EMBED_DIM + x
            for rt0 in range(0, width // 16, 4):
                group = [(rt0 + j, ct)
                         for j in range(4)
                         for ct in range(EMBED_DIM // 16)]
                vals = [
                    plsc.load_gather(
                        in_bufs[buf],
                        [x + 16 * ct, iota16 + 16 * rt])
                    for rt, ct in group
                ]
                for v, (rt, ct) in zip(vals, group):
                    dst = xd + (rt * 16 * EMBED_DIM + 16 * ct)
                    plsc.store_scatter(tr_bufs[buf], [dst], v)

    def writeback(v0, buf, width):
        pltpu.async_copy(
            tr_bufs[buf],
            out_hbm.at[pl.ds(v0 * EMBED_DIM, EMBED_DIM * width)], osem)

    def drain_out(buf, width):
        pltpu.make_async_copy(
            tr_bufs[buf],
            out_hbm.at[pl.ds(0, EMBED_DIM * width)], osem).wait()

    load(wid * TW, 0, TW)

    NI = (NFULL + NW - 1) // NW + 1

    @pl.loop(0, NI // 2)
    def _pair(pr):
        for t in range(2):
            i = 2 * pr + t
            win = wid + NW * i
            buf = t

            @pl.when(wid + NW * (i + 1) < NFULL)
            def _prefetch():
                load((wid + NW * (i + 1)) * TW, 1 - buf, TW)

            @pl.when(win < NFULL)
            def _body():
                drain_in(buf, TW)

                @pl.when(i >= 2)
                def _free():
                    drain_out(buf, TW)

                transpose(buf, TW)
                writeback(win * TW, buf, TW)

    drain_out(0, TW)
    drain_out(1, TW)


@functools.partial(
    pl.kernel,
    out_type=jax.ShapeDtypeStruct((TP * NW, 1024), jnp.float32),
    mesh=_mesh,
    scratch_types=[
        pltpu.VMEM((2, HWIN, BBLK), jnp.int32),
        pltpu.VMEM((ROWS, EMBED_DIM), jnp.float32),
        pltpu.VMEM((ROWS, EMBED_DIM), jnp.float32),
        pltpu.VMEM(((HWIN * EMBED_DIM // 8) * 1024,), jnp.float32),
        pltpu.VMEM(((HWIN * EMBED_DIM // 8) * 1024,), jnp.float32),
        pltpu.SemaphoreType.DMA,
        pltpu.SemaphoreType.DMA,
        pltpu.SemaphoreType.DMA,
    ],
    compiler_params=pltpu.CompilerParams(
        use_tc_tiling_on_sc=False, needs_layout_passes=False),
)
def _emb_lookup(idxT_hbm, table_hbm, out_hbm, idx_v, rows_v0, rows_v1,
                outv0, outv1, gsem, osem, isem):
    rows_bufs = (rows_v0, rows_v1)
    out_bufs = (outv0, outv1)
    wid = lax.axis_index("s") * NUM_CORES + lax.axis_index("c")
    b0 = wid * BBLK

    def load_idx(win, buf):
        pltpu.async_copy(
            idxT_hbm.at[pl.ds(win * HWIN, HWIN), pl.ds(b0, BBLK)],
            idx_v.at[buf], isem)

    def drain_idx(buf):
        pltpu.make_async_copy(
            idxT_hbm.at[pl.ds(0, HWIN), pl.ds(0, BBLK)],
            idx_v.at[buf], isem).wait()

    def fire_gathers(buf):
        for j in range(HWIN):
            for k in range(2):
                pltpu.async_copy(
                    table_hbm.at[idx_v.at[buf, j, pl.ds(64 * k, 64)]],
                    rows_bufs[buf].at[pl.ds(j * BBLK + 64 * k, 64)],
                    gsem,
                )

    def drain_gathers(buf):
        pltpu.make_async_copy(
            table_hbm.at[pl.ds(0, ROWS)], rows_bufs[buf], gsem).wait()

    iota16 = lax.iota(jnp.int32, 16)

    def transpose(buf):
        @pl.loop(0, 16, unroll=2)
        def _d(d):
            x = (d + iota16) & 15
            dst_x = ((x >> 3) << 10) + ((x & 7) << 7) + iota16
            for h in range(HWIN):
                for bt0 in range(0, BBLK // 16, 4):
                    group = [(bt0 + j, k0)
                             for j in range(4)
                             for k0 in range(EMBED_DIM // 16)]
                    vals = [
                        plsc.load_gather(
                            rows_bufs[buf],
                            [iota16 + (h * BBLK + 16 * bt), x + 16 * k0])
                        for bt, k0 in group
                    ]
                    for v, (bt, k0) in zip(vals, group):
                        dst = dst_x + ((h * 4 + 2 * k0) * 1024 + 16 * bt)
                        plsc.store_scatter(out_bufs[buf], [dst], v)

    WTP = HWIN * EMBED_DIM // 8

    def writeback(win, buf):
        for a in range(WTP):
            pltpu.async_copy(
                out_bufs[buf].at[pl.ds(a * 1024, 1024)],
                out_hbm.at[(win * WTP + a) * NW + wid],
                osem)

    def drain_out(buf):
        for a in range(WTP):
            pltpu.make_async_copy(
                out_bufs[buf].at[pl.ds(a * 1024, 1024)], out_hbm.at[0],
                osem).wait()

    load_idx(0, 0)
    drain_idx(0)
    fire_gathers(0)
    load_idx(1, 1)

    @pl.loop(0, NWIN // 2)
    def _pair(pr):
        w0 = 2 * pr
        for t in range(2):
            win = w0 + t
            buf = t
            nbuf = 1 - t

            @pl.when(win + 1 < NWIN)
            def _prefetch():
                drain_idx(nbuf)
                fire_gathers(nbuf)

            drain_gathers(buf)

            @pl.when(win + 2 < NWIN)
            def _iprefetch():
                load_idx(win + 2, buf)

            @pl.when(win >= 2)
            def _free():
                drain_out(buf)

            transpose(buf)
            writeback(win, buf)

    drain_out(0)
    drain_out(1)


@jax.jit
def kernel(w_tensor, table):
    idxT = w_tensor.astype(jnp.int32).T
    table_lin = _table_transpose(table.T)
    tail = table[TAILV:].reshape(TAILW * EMBED_DIM)
    table_lin = table_lin.at[TAILV * EMBED_DIM:].set(tail)
    table_rm = table_lin.reshape(VOCAB, EMBED_DIM)
    out2d = _emb_lookup(idxT, table_rm)
    out4d = out2d.reshape(TP, NW, 8, 128)
    out = out4d.transpose(1, 3, 0, 2).reshape(BATCH, HIST * EMBED_DIM)
    return out.reshape(BATCH, HIST, EMBED_DIM)
